# baseline (speedup 1.0000x reference)
import jax
import jax.numpy as jnp
from jax import lax
from jax.experimental import pallas as pl
from jax.experimental.pallas import tpu as pltpu
from jax.experimental.pallas import tpu_sc as plsc


BATCH = 4
SEQ_LEN = 4096
D_MODEL = 1024

NUM_CORES = 2
NUM_SUBCORES = 16
NUM_WORKERS = NUM_CORES * NUM_SUBCORES
ROWS_PER_WORKER = SEQ_LEN // NUM_WORKERS
CHUNK = 16
CHUNKS_PER_WORKER = ROWS_PER_WORKER // CHUNK
LANES = 16
CHUNK_WORDS = CHUNK * D_MODEL
UNROLL = 16
GROUP_WORDS = UNROLL * LANES
N_GROUPS = CHUNK_WORDS // GROUP_WORDS
NBUF = 4
LOOKAHEAD = 3

_STEPS = [(k, b) for k in range(CHUNKS_PER_WORKER) for b in range(BATCH)]


def _sc_body(pos_hbm, x_hbm, out_hbm,
             tbuf0, tbuf1, xb0, xb1, xb2, xb3,
             sem_t0, sem_t1, si0, si1, si2, si3, so0, so1, so2, so3):
    c = lax.axis_index("c")
    s = lax.axis_index("s")
    wid = s * NUM_CORES + c
    base = wid * ROWS_PER_WORKER * D_MODEL

    tbufs = [tbuf0, tbuf1]
    sems_t = [sem_t0, sem_t1]
    xbufs = [xb0, xb1, xb2, xb3]
    sems_in = [si0, si1, si2, si3]
    sems_out = [so0, so1, so2, so3]

    def toff(k):
        return base + k * CHUNK_WORDS

    def xoff(k, b):
        return b * SEQ_LEN * D_MODEL + toff(k)

    def add_chunk(xb, tb):
        def grp_body(g, _):
            gbase = g * GROUP_WORDS
            for u in range(UNROLL):
                off = gbase + u * LANES
                v = tb[pl.ds(off, LANES)]
                plsc.addupdate(xb.at[pl.ds(off, LANES)], v)
            return 0

        lax.fori_loop(0, N_GROUPS, grp_body, 0)

    n_steps = len(_STEPS)
    t_handles = [None, None]
    in_handles = [None] * NBUF
    out_handles = [None] * NBUF

    def issue_gather(j):
        kj, bj = _STEPS[j]
        bufj = j % NBUF
        if out_handles[bufj] is not None:
            out_handles[bufj].wait()
            out_handles[bufj] = None
        in_handles[bufj] = pltpu.async_copy(
            x_hbm.at[pl.ds(xoff(kj, bj), CHUNK_WORDS)],
            xbufs[bufj], sems_in[bufj])

    t_handles[0] = pltpu.async_copy(
        pos_hbm.at[pl.ds(toff(0), CHUNK_WORDS)], tbufs[0], sems_t[0])
    for j in range(min(LOOKAHEAD, n_steps)):
        issue_gather(j)

    for i, (k, b) in enumerate(_STEPS):
        cur = i % NBUF
        if b == 0 and k + 1 < CHUNKS_PER_WORKER:
            nk = k + 1
            t_handles[nk % 2] = pltpu.async_copy(
                pos_hbm.at[pl.ds(toff(nk), CHUNK_WORDS)],
                tbufs[nk % 2], sems_t[nk % 2])
        if i + LOOKAHEAD < n_steps:
            issue_gather(i + LOOKAHEAD)
        if b == 0:
            t_handles[k % 2].wait()
            t_handles[k % 2] = None
        in_handles[cur].wait()
        add_chunk(xbufs[cur], tbufs[k % 2])
        out_handles[cur] = pltpu.async_copy(
            xbufs[cur], out_hbm.at[pl.ds(xoff(k, b), CHUNK_WORDS)],
            sems_out[cur])

    for h in out_handles:
        if h is not None:
            h.wait()


def kernel(x, pos_table):
    xf = x.reshape(BATCH * SEQ_LEN * D_MODEL)
    posf = pos_table.reshape(SEQ_LEN * D_MODEL)
    mesh = plsc.VectorSubcoreMesh(core_axis_name="c", subcore_axis_name="s")
    out = pl.kernel(
        _sc_body,
        out_type=jax.ShapeDtypeStruct((BATCH * SEQ_LEN * D_MODEL,), x.dtype),
        mesh=mesh,
        scratch_types=(
            [pltpu.VMEM((CHUNK_WORDS,), jnp.float32)] * 2
            + [pltpu.VMEM((CHUNK_WORDS,), jnp.float32)] * NBUF
            + [pltpu.SemaphoreType.DMA] * (2 + 2 * NBUF)
        ),
    )(posf, xf)
    return out.reshape(BATCH, SEQ_LEN, D_MODEL)


# speedup vs baseline: 1.3743x; 1.3743x over previous
import jax
import jax.numpy as jnp
from jax import lax
from jax.experimental import pallas as pl
from jax.experimental.pallas import tpu as pltpu
from jax.experimental.pallas import tpu_sc as plsc


BATCH = 4
SEQ_LEN = 4096
D_MODEL = 1024

NUM_CORES = 2
NUM_SUBCORES = 16
NUM_WORKERS = NUM_CORES * NUM_SUBCORES
ROWS_PER_WORKER = SEQ_LEN // NUM_WORKERS
CHUNK = 16
CHUNKS_PER_WORKER = ROWS_PER_WORKER // CHUNK
LANES = 16
N_STEPS = CHUNKS_PER_WORKER * BATCH


def _sc_body(pos_hbm, x_hbm, out_hbm,
             tbuf0, tbuf1, xbuf0, xbuf1,
             sem_t0, sem_t1, sem_in0, sem_in1, sem_out0, sem_out1):
    c_ax = lax.axis_index("c")
    s_ax = lax.axis_index("s")
    wid = s_ax * NUM_CORES + c_ax
    base = wid * ROWS_PER_WORKER

    def toff(k):
        return base + k * CHUNK

    def xoff(k, b):
        return b * SEQ_LEN + toff(k)

    def step_off(i):
        return xoff(i // BATCH, i % BATCH)

    def issue_t(k, tb, sem):
        pltpu.async_copy(pos_hbm.at[pl.ds(toff(k), CHUNK)], tb, sem)

    def wait_t(k, tb, sem):
        pltpu.make_async_copy(pos_hbm.at[pl.ds(toff(k), CHUNK)], tb, sem).wait()

    def add_chunk(xb, tb):
        def row_body(r, _):
            for u in range(D_MODEL // LANES):
                off = u * LANES
                v = tb[r, pl.ds(off, LANES)]
                plsc.addupdate(xb.at[r, pl.ds(off, LANES)], v)
            return 0

        lax.fori_loop(0, CHUNK, row_body, 0)

    def substep(i, tb, xb_cur, sem_in_cur, sem_out_cur,
                xb_oth, sem_in_oth, sem_out_oth):
        off_cur = step_off(i)
        @pl.when(i >= 1)
        def _():
            pltpu.make_async_copy(
                xb_oth, out_hbm.at[pl.ds(step_off(i - 1), CHUNK)],
                sem_out_oth).wait()

        @pl.when(i + 1 < N_STEPS)
        def _():
            pltpu.async_copy(
                x_hbm.at[pl.ds(step_off(i + 1), CHUNK)], xb_oth, sem_in_oth)

        pltpu.make_async_copy(
            x_hbm.at[pl.ds(off_cur, CHUNK)], xb_cur, sem_in_cur).wait()
        add_chunk(xb_cur, tb)
        pltpu.async_copy(xb_cur, out_hbm.at[pl.ds(off_cur, CHUNK)], sem_out_cur)

    def chunk_body(k, tb, sem_t):
        wait_t(k, tb, sem_t)

        def m_body(m, _):
            i = k * BATCH + 2 * m
            substep(i, tb, xbuf0, sem_in0, sem_out0,
                    xbuf1, sem_in1, sem_out1)
            substep(i + 1, tb, xbuf1, sem_in1, sem_out1,
                    xbuf0, sem_in0, sem_out0)
            return 0

        lax.fori_loop(0, BATCH // 2, m_body, 0)

        @pl.when(k + 2 < CHUNKS_PER_WORKER)
        def _():
            issue_t(k + 2, tb, sem_t)

    issue_t(0, tbuf0, sem_t0)
    issue_t(1, tbuf1, sem_t1)
    pltpu.async_copy(x_hbm.at[pl.ds(step_off(0), CHUNK)], xbuf0, sem_in0)

    def jj_body(jj, _):
        chunk_body(2 * jj, tbuf0, sem_t0)
        chunk_body(2 * jj + 1, tbuf1, sem_t1)
        return 0

    lax.fori_loop(0, CHUNKS_PER_WORKER // 2, jj_body, 0)

    pltpu.make_async_copy(
        xbuf1, out_hbm.at[pl.ds(step_off(N_STEPS - 1), CHUNK)], sem_out1).wait()


def kernel(x, pos_table):
    xf = x.reshape(BATCH * SEQ_LEN, D_MODEL)
    mesh = plsc.VectorSubcoreMesh(core_axis_name="c", subcore_axis_name="s")
    out = pl.kernel(
        _sc_body,
        out_type=jax.ShapeDtypeStruct((BATCH * SEQ_LEN, D_MODEL), x.dtype),
        mesh=mesh,
        scratch_types=(
            [pltpu.VMEM((CHUNK, D_MODEL), jnp.float32)] * 4
            + [pltpu.SemaphoreType.DMA] * 6
        ),
    )(pos_table, xf)
    return out.reshape(BATCH, SEQ_LEN, D_MODEL)


# speedup vs baseline: 1.3982x; 1.0174x over previous
import jax
import jax.numpy as jnp
from jax import lax
from jax.experimental import pallas as pl
from jax.experimental.pallas import tpu as pltpu
from jax.experimental.pallas import tpu_sc as plsc


BATCH = 4
SEQ_LEN = 4096
D_MODEL = 1024

NUM_CORES = 2
NUM_SUBCORES = 16
NUM_WORKERS = NUM_CORES * NUM_SUBCORES
ROWS_PER_WORKER = SEQ_LEN // NUM_WORKERS
CHUNK = 8
N_STEPS = ROWS_PER_WORKER // CHUNK
LANES = 16
NXB = 3
LOOKAHEAD = 2


def _sc_body(pos_hbm, x_hbm, out_hbm,
             tbuf0, tbuf1, xb0, xb1, xb2,
             sem_t0, sem_t1, si0, si1, si2, so0, so1, so2):
    c_ax = lax.axis_index("c")
    s_ax = lax.axis_index("s")
    wid = s_ax * NUM_CORES + c_ax
    base = wid * ROWS_PER_WORKER

    tbufs = [tbuf0, tbuf1]
    sems_t = [sem_t0, sem_t1]
    xbufs = [xb0, xb1, xb2]
    sems_in = [si0, si1, si2]
    sems_out = [so0, so1, so2]

    def srow(i):
        return base + i * CHUNK

    def t_issue(i, tb, sem):
        pltpu.async_copy(pos_hbm.at[pl.ds(srow(i), CHUNK)], tb, sem)

    def t_wait(i, tb, sem):
        pltpu.make_async_copy(
            pos_hbm.at[pl.ds(srow(i), CHUNK)], tb, sem).wait()

    def x_issue(i, xb, sem):
        pltpu.async_copy(
            x_hbm.at[:, pl.ds(srow(i), CHUNK), :], xb, sem)

    def x_wait(i, xb, sem):
        pltpu.make_async_copy(
            x_hbm.at[:, pl.ds(srow(i), CHUNK), :], xb, sem).wait()

    def o_issue(i, xb, sem):
        pltpu.async_copy(
            xb, out_hbm.at[:, pl.ds(srow(i), CHUNK), :], sem)

    def o_wait(i, xb, sem):
        pltpu.make_async_copy(
            xb, out_hbm.at[:, pl.ds(srow(i), CHUNK), :], sem).wait()

    def add_step(xb, tb):
        def b_body(b, _):
            def r_body(r, _):
                for u in range(D_MODEL // LANES):
                    off = u * LANES
                    v = tb[r, pl.ds(off, LANES)]
                    plsc.addupdate(xb.at[b, r, pl.ds(off, LANES)], v)
                return 0

            lax.fori_loop(0, CHUNK, r_body, 0)
            return 0

        lax.fori_loop(0, BATCH, b_body, 0)

    def substep(i, tb, sem_t, xb_cur, sem_in_cur, sem_out_cur,
                xb_pre, sem_in_pre, sem_out_pre):
        @pl.when(i + LOOKAHEAD < N_STEPS)
        def _():
            @pl.when(i + LOOKAHEAD - NXB >= 0)
            def _():
                o_wait(i + LOOKAHEAD - NXB, xb_pre, sem_out_pre)

            x_issue(i + LOOKAHEAD, xb_pre, sem_in_pre)

        t_wait(i, tb, sem_t)
        x_wait(i, xb_cur, sem_in_cur)
        add_step(xb_cur, tb)
        o_issue(i, xb_cur, sem_out_cur)

        @pl.when(i + 2 < N_STEPS)
        def _():
            t_issue(i + 2, tb, sem_t)

    t_issue(0, tbuf0, sem_t0)
    t_issue(1, tbuf1, sem_t1)
    x_issue(0, xbufs[0], sems_in[0])
    x_issue(1, xbufs[1], sems_in[1])

    def tri_body(j, _):
        for t in range(6):
            i = 6 * j + t

            @pl.when(i < N_STEPS)
            def _(i=i, t=t):
                substep(i, tbufs[t % 2], sems_t[t % 2],
                        xbufs[t % 3], sems_in[t % 3], sems_out[t % 3],
                        xbufs[(t + LOOKAHEAD) % 3],
                        sems_in[(t + LOOKAHEAD) % 3],
                        sems_out[(t + LOOKAHEAD) % 3])

        return 0

    lax.fori_loop(0, (N_STEPS + 5) // 6, tri_body, 0)

    for d in range(NXB):
        i = N_STEPS - NXB + d
        o_wait(i, xbufs[i % 3], sems_out[i % 3])


def kernel(x, pos_table):
    mesh = plsc.VectorSubcoreMesh(core_axis_name="c", subcore_axis_name="s")
    out = pl.kernel(
        _sc_body,
        out_type=jax.ShapeDtypeStruct((BATCH, SEQ_LEN, D_MODEL), x.dtype),
        mesh=mesh,
        scratch_types=(
            [pltpu.VMEM((CHUNK, D_MODEL), jnp.float32)] * 2
            + [pltpu.VMEM((BATCH, CHUNK, D_MODEL), jnp.float32)] * NXB
            + [pltpu.SemaphoreType.DMA] * 8
        ),
    )(pos_table, x)
    return out


# speedup vs baseline: 1.9665x; 1.4064x over previous
import jax
import jax.numpy as jnp
from jax import lax
from jax.experimental import pallas as pl
from jax.experimental.pallas import tpu as pltpu
from jax.experimental.pallas import tpu_sc as plsc


BATCH = 4
SEQ_LEN = 4096
D_MODEL = 1024

TC_ROWS = 3072
SC_ROWS = SEQ_LEN - TC_ROWS
TC_BLOCK_S = 1024

NUM_CORES = 2
NUM_SUBCORES = 16
NUM_WORKERS = NUM_CORES * NUM_SUBCORES
ROWS_PER_WORKER = SC_ROWS // NUM_WORKERS
CHUNK = 8
N_STEPS = ROWS_PER_WORKER // CHUNK
LANES = 16
NXB = 3
LOOKAHEAD = 2


def _tc_block(x_ref, pos_ref, o_ref):
    o_ref[...] = x_ref[...] + pos_ref[...][None]


def _sc_body(pos_hbm, x_hbm, out_hbm,
             tbuf0, tbuf1, xb0, xb1, xb2,
             sem_t0, sem_t1, si0, si1, si2, so0, so1, so2):
    c_ax = lax.axis_index("c")
    s_ax = lax.axis_index("s")
    wid = s_ax * NUM_CORES + c_ax
    base = wid * ROWS_PER_WORKER

    tbufs = [tbuf0, tbuf1]
    sems_t = [sem_t0, sem_t1]
    xbufs = [xb0, xb1, xb2]
    sems_in = [si0, si1, si2]
    sems_out = [so0, so1, so2]

    def srow(i):
        return base + i * CHUNK

    def t_issue(i):
        pltpu.async_copy(
            pos_hbm.at[pl.ds(TC_ROWS + srow(i), CHUNK)],
            tbufs[i % 2], sems_t[i % 2])

    def add_step(xb, tb):
        def b_body(b, _):
            def r_body(r, _):
                for u in range(D_MODEL // LANES):
                    off = u * LANES
                    v = tb[r, pl.ds(off, LANES)]
                    plsc.addupdate(xb.at[b, r, pl.ds(off, LANES)], v)
                return 0

            lax.fori_loop(0, CHUNK, r_body, 0)
            return 0

        lax.fori_loop(0, BATCH, b_body, 0)

    t_issue(0)
    t_issue(1)
    in_handles = [None] * NXB
    out_handles = [None] * NXB
    for j in range(min(LOOKAHEAD, N_STEPS)):
        in_handles[j % NXB] = pltpu.async_copy(
            x_hbm.at[:, pl.ds(TC_ROWS + srow(j), CHUNK), :],
            xbufs[j % NXB], sems_in[j % NXB])

    for i in range(N_STEPS):
        cur = i % NXB
        j = i + LOOKAHEAD
        if j < N_STEPS:
            slot = j % NXB
            if out_handles[slot] is not None:
                out_handles[slot].wait()
                out_handles[slot] = None
            in_handles[slot] = pltpu.async_copy(
                x_hbm.at[:, pl.ds(TC_ROWS + srow(j), CHUNK), :],
                xbufs[slot], sems_in[slot])
        pltpu.make_async_copy(
            pos_hbm.at[pl.ds(TC_ROWS + srow(i), CHUNK)],
            tbufs[i % 2], sems_t[i % 2]).wait()
        in_handles[cur].wait()
        add_step(xbufs[cur], tbufs[i % 2])
        if i + 2 < N_STEPS:
            t_issue(i + 2)
        out_handles[cur] = pltpu.async_copy(
            xbufs[cur], out_hbm.at[:, pl.ds(srow(i), CHUNK), :],
            sems_out[cur])

    for h in out_handles:
        if h is not None:
            h.wait()


def kernel(x, pos_table):
    tc_out = pl.pallas_call(
        _tc_block,
        grid=(TC_ROWS // TC_BLOCK_S, BATCH),
        in_specs=[
            pl.BlockSpec((1, TC_BLOCK_S, D_MODEL), lambda s, b: (b, s, 0)),
            pl.BlockSpec((TC_BLOCK_S, D_MODEL), lambda s, b: (s, 0)),
        ],
        out_specs=pl.BlockSpec((1, TC_BLOCK_S, D_MODEL), lambda s, b: (b, s, 0)),
        out_shape=jax.ShapeDtypeStruct((BATCH, TC_ROWS, D_MODEL), x.dtype),
    )(x, pos_table)

    mesh = plsc.VectorSubcoreMesh(core_axis_name="c", subcore_axis_name="s")
    sc_out = pl.kernel(
        _sc_body,
        out_type=jax.ShapeDtypeStruct((BATCH, SC_ROWS, D_MODEL), x.dtype),
        mesh=mesh,
        scratch_types=(
            [pltpu.VMEM((CHUNK, D_MODEL), jnp.float32)] * 2
            + [pltpu.VMEM((BATCH, CHUNK, D_MODEL), jnp.float32)] * NXB
            + [pltpu.SemaphoreType.DMA] * 8
        ),
    )(pos_table, x)

    return jnp.concatenate([tc_out, sc_out], axis=1)


# speedup vs baseline: 2.3062x; 1.1727x over previous
import jax
import jax.numpy as jnp
from jax import lax
from jax.experimental import pallas as pl
from jax.experimental.pallas import tpu as pltpu
from jax.experimental.pallas import tpu_sc as plsc


BATCH = 4
SEQ_LEN = 4096
D_MODEL = 1024

TC_ROWS = 3072
SC_ROWS = SEQ_LEN - TC_ROWS
TC_BLOCK_S = 1024

NUM_CORES = 2
NUM_SUBCORES = 16
NUM_WORKERS = NUM_CORES * NUM_SUBCORES
ROWS_PER_WORKER = SC_ROWS // NUM_WORKERS
CHUNK = 8
N_STEPS = ROWS_PER_WORKER // CHUNK
LANES = 16
NXB = 3
LOOKAHEAD = 2


def _tc_block(x_ref, pos_ref, o_ref):
    o_ref[...] = x_ref[...] + pos_ref[...][None]


def _sc_body(pos_hbm, x_hbm, out_hbm,
             tbuf0, tbuf1, xb0, xb1, xb2,
             sem_t0, sem_t1, si0, si1, si2, so0, so1, so2):
    c_ax = lax.axis_index("c")
    s_ax = lax.axis_index("s")
    wid = s_ax * NUM_CORES + c_ax
    base = wid * ROWS_PER_WORKER

    tbufs = [tbuf0, tbuf1]
    sems_t = [sem_t0, sem_t1]
    xbufs = [xb0, xb1, xb2]
    sems_in = [si0, si1, si2]
    sems_out = [so0, so1, so2]

    def srow(i):
        return base + i * CHUNK

    def t_issue(i):
        pltpu.async_copy(
            pos_hbm.at[pl.ds(TC_ROWS + srow(i), CHUNK)],
            tbufs[i % 2], sems_t[i % 2])

    def add_step(xb, tb):
        def b_body(b, _):
            def r_body(r, _):
                for u in range(D_MODEL // LANES):
                    off = u * LANES
                    v = tb[r, pl.ds(off, LANES)]
                    plsc.addupdate(xb.at[b, r, pl.ds(off, LANES)], v)
                return 0

            lax.fori_loop(0, CHUNK, r_body, 0)
            return 0

        lax.fori_loop(0, BATCH, b_body, 0)

    t_issue(0)
    t_issue(1)
    in_handles = [None] * NXB
    out_handles = [None] * NXB
    for j in range(min(LOOKAHEAD, N_STEPS)):
        in_handles[j % NXB] = pltpu.async_copy(
            x_hbm.at[:, pl.ds(TC_ROWS + srow(j), CHUNK), :],
            xbufs[j % NXB], sems_in[j % NXB])

    for i in range(N_STEPS):
        cur = i % NXB
        j = i + LOOKAHEAD
        if j < N_STEPS:
            slot = j % NXB
            if out_handles[slot] is not None:
                out_handles[slot].wait()
                out_handles[slot] = None
            in_handles[slot] = pltpu.async_copy(
                x_hbm.at[:, pl.ds(TC_ROWS + srow(j), CHUNK), :],
                xbufs[slot], sems_in[slot])
        pltpu.make_async_copy(
            pos_hbm.at[pl.ds(TC_ROWS + srow(i), CHUNK)],
            tbufs[i % 2], sems_t[i % 2]).wait()
        in_handles[cur].wait()
        add_step(xbufs[cur], tbufs[i % 2])
        if i + 2 < N_STEPS:
            t_issue(i + 2)
        out_handles[cur] = pltpu.async_copy(
            xbufs[cur], out_hbm.at[:, pl.ds(TC_ROWS + srow(i), CHUNK), :],
            sems_out[cur])

    for h in out_handles:
        if h is not None:
            h.wait()


def _tc_block_alias(x_ref, pos_ref, sc_ref, o_ref):
    del sc_ref
    o_ref[...] = x_ref[...] + pos_ref[...][None]


def kernel(x, pos_table):
    mesh = plsc.VectorSubcoreMesh(core_axis_name="c", subcore_axis_name="s")
    sc_full = pl.kernel(
        _sc_body,
        out_type=jax.ShapeDtypeStruct((BATCH, SEQ_LEN, D_MODEL), x.dtype),
        mesh=mesh,
        scratch_types=(
            [pltpu.VMEM((CHUNK, D_MODEL), jnp.float32)] * 2
            + [pltpu.VMEM((BATCH, CHUNK, D_MODEL), jnp.float32)] * NXB
            + [pltpu.SemaphoreType.DMA] * 8
        ),
    )(pos_table, x)

    return pl.pallas_call(
        _tc_block_alias,
        grid=(TC_ROWS // TC_BLOCK_S, BATCH),
        in_specs=[
            pl.BlockSpec((1, TC_BLOCK_S, D_MODEL), lambda s, b: (b, s, 0)),
            pl.BlockSpec((TC_BLOCK_S, D_MODEL), lambda s, b: (s, 0)),
            pl.BlockSpec(memory_space=pl.ANY),
        ],
        out_specs=pl.BlockSpec((1, TC_BLOCK_S, D_MODEL), lambda s, b: (b, s, 0)),
        out_shape=jax.ShapeDtypeStruct((BATCH, SEQ_LEN, D_MODEL), x.dtype),
        input_output_aliases={2: 0},
    )(x, pos_table, sc_full)


# speedup vs baseline: 2.7085x; 1.1745x over previous
import jax
import jax.numpy as jnp
from jax import lax
from jax.experimental import pallas as pl
from jax.experimental.pallas import tpu as pltpu
from jax.experimental.pallas import tpu_sc as plsc


BATCH = 4
SEQ_LEN = 4096
D_MODEL = 1024

TC_ROWS = 3584
SC_ROWS = SEQ_LEN - TC_ROWS
TC_BLOCK_S = 1792

NUM_CORES = 2
NUM_SUBCORES = 16
NUM_WORKERS = NUM_CORES * NUM_SUBCORES
ROWS_PER_WORKER = SC_ROWS // NUM_WORKERS
CHUNK = 8
N_STEPS = ROWS_PER_WORKER // CHUNK
LANES = 16
NXB = 3
LOOKAHEAD = 2


def _tc_block(x_ref, pos_ref, o_ref):
    o_ref[...] = x_ref[...] + pos_ref[...][None]


def _sc_body(pos_hbm, x_hbm, out_hbm,
             tbuf0, tbuf1, xb0, xb1, xb2,
             sem_t0, sem_t1, si0, si1, si2, so0, so1, so2):
    c_ax = lax.axis_index("c")
    s_ax = lax.axis_index("s")
    wid = s_ax * NUM_CORES + c_ax
    base = wid * ROWS_PER_WORKER

    tbufs = [tbuf0, tbuf1]
    sems_t = [sem_t0, sem_t1]
    xbufs = [xb0, xb1, xb2]
    sems_in = [si0, si1, si2]
    sems_out = [so0, so1, so2]

    def srow(i):
        return base + i * CHUNK

    def t_issue(i):
        pltpu.async_copy(
            pos_hbm.at[pl.ds(TC_ROWS + srow(i), CHUNK)],
            tbufs[i % 2], sems_t[i % 2])

    def add_step(xb, tb):
        def b_body(b, _):
            def r_body(r, _):
                for u in range(D_MODEL // LANES):
                    off = u * LANES
                    v = tb[r, pl.ds(off, LANES)]
                    plsc.addupdate(xb.at[b, r, pl.ds(off, LANES)], v)
                return 0

            lax.fori_loop(0, CHUNK, r_body, 0)
            return 0

        lax.fori_loop(0, BATCH, b_body, 0)

    t_issue(0)
    t_issue(1)
    in_handles = [None] * NXB
    out_handles = [None] * NXB
    for j in range(min(LOOKAHEAD, N_STEPS)):
        in_handles[j % NXB] = pltpu.async_copy(
            x_hbm.at[:, pl.ds(TC_ROWS + srow(j), CHUNK), :],
            xbufs[j % NXB], sems_in[j % NXB])

    for i in range(N_STEPS):
        cur = i % NXB
        j = i + LOOKAHEAD
        if j < N_STEPS:
            slot = j % NXB
            if out_handles[slot] is not None:
                out_handles[slot].wait()
                out_handles[slot] = None
            in_handles[slot] = pltpu.async_copy(
                x_hbm.at[:, pl.ds(TC_ROWS + srow(j), CHUNK), :],
                xbufs[slot], sems_in[slot])
        pltpu.make_async_copy(
            pos_hbm.at[pl.ds(TC_ROWS + srow(i), CHUNK)],
            tbufs[i % 2], sems_t[i % 2]).wait()
        in_handles[cur].wait()
        add_step(xbufs[cur], tbufs[i % 2])
        if i + 2 < N_STEPS:
            t_issue(i + 2)
        out_handles[cur] = pltpu.async_copy(
            xbufs[cur], out_hbm.at[:, pl.ds(TC_ROWS + srow(i), CHUNK), :],
            sems_out[cur])

    for h in out_handles:
        if h is not None:
            h.wait()


def _tc_block_alias(x_ref, pos_ref, sc_ref, o_ref):
    del sc_ref
    o_ref[...] = x_ref[...] + pos_ref[...][None]


def kernel(x, pos_table):
    mesh = plsc.VectorSubcoreMesh(core_axis_name="c", subcore_axis_name="s")
    sc_full = pl.kernel(
        _sc_body,
        out_type=jax.ShapeDtypeStruct((BATCH, SEQ_LEN, D_MODEL), x.dtype),
        mesh=mesh,
        scratch_types=(
            [pltpu.VMEM((CHUNK, D_MODEL), jnp.float32)] * 2
            + [pltpu.VMEM((BATCH, CHUNK, D_MODEL), jnp.float32)] * NXB
            + [pltpu.SemaphoreType.DMA] * 8
        ),
    )(pos_table, x)

    return pl.pallas_call(
        _tc_block_alias,
        grid=(TC_ROWS // TC_BLOCK_S, BATCH),
        in_specs=[
            pl.BlockSpec((1, TC_BLOCK_S, D_MODEL), lambda s, b: (b, s, 0)),
            pl.BlockSpec((TC_BLOCK_S, D_MODEL), lambda s, b: (s, 0)),
            pl.BlockSpec(memory_space=pl.ANY),
        ],
        out_specs=pl.BlockSpec((1, TC_BLOCK_S, D_MODEL), lambda s, b: (b, s, 0)),
        out_shape=jax.ShapeDtypeStruct((BATCH, SEQ_LEN, D_MODEL), x.dtype),
        input_output_aliases={2: 0},
    )(x, pos_table, sc_full)
